# Initial kernel scaffold; baseline (speedup 1.0000x reference)
import jax
import jax.numpy as jnp
from jax.experimental import pallas as pl


def kernel(utterance, W):
    raise NotImplementedError("write your pallas kernel here")


# speedup vs baseline: 2.0565x; 2.0565x over previous
import functools

import jax
import jax.numpy as jnp
from jax import lax
from jax.experimental import pallas as pl
from jax.experimental.pallas import tpu as pltpu
from jax.experimental.pallas import tpu_sc as plsc

VOCAB = 100000
E = 128
B = 4096
S = 20
LANES = 16

NC, NS = 2, 16
NW = NC * NS
BPW = B // NW
CB = 16
NCHUNK = BPW // CB
TPC = CB * S
GN = 64
NG = TPC // GN
IDX_ROWS = BPW * S // GN

_mesh = plsc.VectorSubcoreMesh(core_axis_name="c", subcore_axis_name="s")


@functools.partial(
    pl.kernel,
    out_type=jax.ShapeDtypeStruct((B, E), jnp.float32),
    mesh=_mesh,
    scratch_types=[
        pltpu.VMEM((IDX_ROWS, GN), jnp.int32),
        pltpu.VMEM((TPC, E), jnp.float32),
        pltpu.VMEM((TPC, E), jnp.float32),
        pltpu.VMEM((CB, E), jnp.float32),
        pltpu.SemaphoreType.DMA,
        pltpu.SemaphoreType.DMA,
    ],
)
def _sc_embed_sum(utt_hbm, w_hbm, out_hbm, idx_v, rows0, rows1, out_v,
                  sem0, sem1):
    wid = lax.axis_index("s") * NC + lax.axis_index("c")
    pltpu.sync_copy(utt_hbm.at[wid], idx_v)

    bufs = (rows0, rows1)
    sems = (sem0, sem1)

    def fire(c):
        buf = bufs[c % 2]
        sem = sems[c % 2]
        return [
            pltpu.async_copy(
                w_hbm.at[idx_v.at[c * NG + j]],
                buf.at[pl.ds(j * GN, GN)],
                sem,
            )
            for j in range(NG)
        ]

    handles = fire(0)
    for c in range(NCHUNK):
        nxt = fire(c + 1) if c + 1 < NCHUNK else None
        for h in handles:
            h.wait()
        buf = bufs[c % 2]

        def accum(b, _, buf=buf):
            for eb in range(E // LANES):
                col = pl.ds(eb * LANES, LANES)
                acc = buf[b * S, col]
                for s in range(1, S):
                    acc = acc + buf[b * S + s, col]
                out_v[b, col] = acc
            return 0

        lax.fori_loop(0, CB, accum, 0)
        pltpu.sync_copy(out_v, out_hbm.at[pl.ds(wid * BPW + c * CB, CB)])
        handles = nxt


def kernel(utterance, W):
    utt = utterance.astype(jnp.int32).reshape(NW, IDX_ROWS, GN)
    return _sc_embed_sum(utt, W)


# speedup vs baseline: 2.8667x; 1.3940x over previous
import functools

import jax
import jax.numpy as jnp
from jax import lax
from jax.experimental import pallas as pl
from jax.experimental.pallas import tpu as pltpu
from jax.experimental.pallas import tpu_sc as plsc

VOCAB = 100000
E = 128
B = 4096
S = 20
LANES = 16

NC, NS = 2, 16
NW = NC * NS
BPW = B // NW
CB = 16
NCHUNK = BPW // CB
TPC = CB * S
GN = 64
NG = TPC // GN
IDX_ROWS = BPW * S // GN

_mesh = plsc.VectorSubcoreMesh(core_axis_name="c", subcore_axis_name="s")


@functools.partial(
    pl.kernel,
    out_type=jax.ShapeDtypeStruct((B, E), jnp.float32),
    mesh=_mesh,
    scratch_types=[
        pltpu.VMEM((IDX_ROWS, GN), jnp.int32),
        pltpu.VMEM((TPC, E), jnp.float32),
        pltpu.VMEM((TPC, E), jnp.float32),
        pltpu.VMEM((CB, E), jnp.float32),
        pltpu.SemaphoreType.DMA,
        pltpu.SemaphoreType.DMA,
    ],
)
def _sc_embed_sum(utt_hbm, w_hbm, out_hbm, idx_v, rows0, rows1, out_v,
                  sem0, sem1):
    wid = lax.axis_index("s") * NC + lax.axis_index("c")
    pltpu.sync_copy(utt_hbm.at[wid], idx_v)

    bufs = (rows0, rows1)
    sems = (sem0, sem1)

    def fire(c):
        buf = bufs[c % 2]
        sem = sems[c % 2]
        return [
            pltpu.async_copy(
                w_hbm.at[idx_v.at[c * NG + j]],
                buf.at[pl.ds(j * GN, GN)],
                sem,
            )
            for j in range(NG)
        ]

    handles = fire(0)
    for c in range(NCHUNK):
        nxt = fire(c + 1) if c + 1 < NCHUNK else None
        for h in handles:
            h.wait()
        buf = bufs[c % 2]

        def accum(b, _, buf=buf):
            cols = [pl.ds(eb * LANES, LANES) for eb in range(E // LANES)]
            accs = [buf[b * S, col] for col in cols]
            for s in range(1, S):
                row = b * S + s
                accs = [acc + buf[row, col] for acc, col in zip(accs, cols)]
            for col, acc in zip(cols, accs):
                out_v[b, col] = acc
            return 0

        lax.fori_loop(0, CB, accum, 0)
        pltpu.sync_copy(out_v, out_hbm.at[pl.ds(wid * BPW + c * CB, CB)])
        handles = nxt


def kernel(utterance, W):
    utt = utterance.astype(jnp.int32).reshape(NW, IDX_ROWS, GN)
    return _sc_embed_sum(utt, W)


# speedup vs baseline: 2.8815x; 1.0052x over previous
import functools

import jax
import jax.numpy as jnp
from jax import lax
from jax.experimental import pallas as pl
from jax.experimental.pallas import tpu as pltpu
from jax.experimental.pallas import tpu_sc as plsc

VOCAB = 100000
E = 128
B = 4096
S = 20
LANES = 16

NC, NS = 2, 16
NW = NC * NS
BPW = B // NW
CB = 16
NCHUNK = BPW // CB
TPC = CB * S
GN = 64
NG = TPC // GN
IDX_ROWS = BPW * S // GN

_mesh = plsc.VectorSubcoreMesh(core_axis_name="c", subcore_axis_name="s")


@functools.partial(
    pl.kernel,
    out_type=jax.ShapeDtypeStruct((B, E), jnp.float32),
    mesh=_mesh,
    scratch_types=[
        pltpu.VMEM((IDX_ROWS, GN), jnp.int32),
        pltpu.VMEM((TPC, E), jnp.float32),
        pltpu.VMEM((TPC, E), jnp.float32),
        pltpu.VMEM((CB, E), jnp.float32),
        pltpu.VMEM((CB, E), jnp.float32),
        pltpu.SemaphoreType.DMA,
        pltpu.SemaphoreType.DMA,
        pltpu.SemaphoreType.DMA,
    ],
)
def _sc_embed_sum(utt_hbm, w_hbm, out_hbm, idx_v, rows0, rows1, outv0,
                  outv1, sem0, sem1, sem_out):
    wid = lax.axis_index("s") * NC + lax.axis_index("c")
    pltpu.sync_copy(utt_hbm.at[wid], idx_v)

    bufs = (rows0, rows1)
    sems = (sem0, sem1)
    outs = (outv0, outv1)

    def fire(c):
        buf = bufs[c % 2]
        sem = sems[c % 2]
        return [
            pltpu.async_copy(
                w_hbm.at[idx_v.at[c * NG + j]],
                buf.at[pl.ds(j * GN, GN)],
                sem,
            )
            for j in range(NG)
        ]

    handles = fire(0)
    store_handles = [None, None]
    for c in range(NCHUNK):
        nxt = fire(c + 1) if c + 1 < NCHUNK else None
        for h in handles:
            h.wait()
        buf = bufs[c % 2]
        out_v = outs[c % 2]
        if store_handles[c % 2] is not None:
            store_handles[c % 2].wait()

        def accum(b, _, buf=buf, out_v=out_v):
            cols = [pl.ds(eb * LANES, LANES) for eb in range(E // LANES)]
            accs = [buf[b * S, col] for col in cols]
            for s in range(1, S):
                row = b * S + s
                accs = [acc + buf[row, col] for acc, col in zip(accs, cols)]
            for col, acc in zip(cols, accs):
                out_v[b, col] = acc
            return 0

        lax.fori_loop(0, CB, accum, 0)
        store_handles[c % 2] = pltpu.async_copy(
            out_v, out_hbm.at[pl.ds(wid * BPW + c * CB, CB)], sem_out)
        handles = nxt
    for h in store_handles:
        h.wait()


def kernel(utterance, W):
    utt = utterance.astype(jnp.int32).reshape(NW, IDX_ROWS, GN)
    return _sc_embed_sum(utt, W)
